# baseline (speedup 1.0000x reference)
import functools

import jax
import jax.numpy as jnp
from jax import lax
from jax.experimental import pallas as pl
from jax.experimental.pallas import tpu as pltpu
from jax.experimental.pallas import tpu_sc as plsc

NUM_Q = 100000
D = 256
CH = 200
NCHUNKS = NUM_Q // CH
NW = 32
PER_W = -(-NCHUNKS // NW)
LAST = NCHUNKS - 1


def _sc_body(e_in, p_in, e_out, p_out, buf, lsem, ssem):
    wid = lax.axis_index("s") * 2 + lax.axis_index("c")

    def chunk_ds(k):
        j = jnp.minimum(wid + k * NW, LAST)
        return pl.ds(j * CH, CH)

    def copy_table(src, dst):
        def load(k, slot):
            c = pltpu.make_async_copy(src.at[chunk_ds(k)], buf.at[slot],
                                      lsem.at[slot])
            c.start()
            return c

        def store(k, slot):
            c = pltpu.make_async_copy(buf.at[slot], dst.at[chunk_ds(k)],
                                      ssem.at[slot])
            c.start()
            return c

        loads = [None] * PER_W
        stores = [None] * PER_W
        loads[0] = load(0, 0)
        for k in range(PER_W):
            slot = k % 2
            if k + 1 < PER_W:
                if k - 1 >= 0:
                    stores[k - 1].wait()
                loads[k + 1] = load(k + 1, (k + 1) % 2)
            loads[k].wait()
            stores[k] = store(k, slot)
        stores[PER_W - 2].wait()
        stores[PER_W - 1].wait()

    copy_table(e_in, e_out)
    copy_table(p_in, p_out)


def kernel(batch_size, query_embed_weight, query_pos_weight):
    out = jax.ShapeDtypeStruct((NUM_Q, D), jnp.float32)
    mesh = plsc.VectorSubcoreMesh(core_axis_name="c", subcore_axis_name="s")
    k = functools.partial(
        pl.kernel,
        out_type=[out, out],
        mesh=mesh,
        scratch_types=[
            pltpu.VMEM((2, CH, D), jnp.float32),
            pltpu.SemaphoreType.DMA((2,)),
            pltpu.SemaphoreType.DMA((2,)),
        ],
    )(_sc_body)
    query_embed, query_pos = k(query_embed_weight, query_pos_weight)
    return (query_embed, query_pos)


# speedup vs baseline: 1.0878x; 1.0878x over previous
import functools

import jax
import jax.numpy as jnp
from jax import lax
from jax.experimental import pallas as pl
from jax.experimental.pallas import tpu as pltpu
from jax.experimental.pallas import tpu_sc as plsc

NUM_Q = 100000
D = 256

CH = 200
NCHUNKS = NUM_Q // CH
NW = 32
PER_W = -(-NCHUNKS // NW)
LAST = NCHUNKS - 1


def _sc_body(src, dst, buf, lsem, ssem):
    wid = lax.axis_index("s") * 2 + lax.axis_index("c")

    def chunk_ds(k):
        j = jnp.minimum(wid + k * NW, LAST)
        return pl.ds(j * CH, CH)

    def load(k, slot):
        c = pltpu.make_async_copy(src.at[chunk_ds(k)], buf.at[slot],
                                  lsem.at[slot])
        c.start()
        return c

    def store(k, slot):
        c = pltpu.make_async_copy(buf.at[slot], dst.at[chunk_ds(k)],
                                  ssem.at[slot])
        c.start()
        return c

    loads = [None] * PER_W
    stores = [None] * PER_W
    loads[0] = load(0, 0)
    for k in range(PER_W):
        slot = k % 2
        if k + 1 < PER_W:
            if k - 1 >= 0:
                stores[k - 1].wait()
            loads[k + 1] = load(k + 1, (k + 1) % 2)
        loads[k].wait()
        stores[k] = store(k, slot)
    stores[PER_W - 2].wait()
    stores[PER_W - 1].wait()


def _sc_copy(table):
    out = jax.ShapeDtypeStruct((NUM_Q, D), jnp.float32)
    mesh = plsc.VectorSubcoreMesh(core_axis_name="c", subcore_axis_name="s")
    k = functools.partial(
        pl.kernel,
        out_type=out,
        mesh=mesh,
        scratch_types=[
            pltpu.VMEM((2, CH, D), jnp.float32),
            pltpu.SemaphoreType.DMA((2,)),
            pltpu.SemaphoreType.DMA((2,)),
        ],
    )(_sc_body)
    return k(table)


BLOCK = 5000
GRID = NUM_Q // BLOCK


def _tc_body(src, dst):
    dst[...] = src[...]


def _tc_copy(table):
    out = jax.ShapeDtypeStruct((NUM_Q, D), jnp.float32)
    spec = pl.BlockSpec((BLOCK, D), lambda i: (i, 0))
    return pl.pallas_call(
        _tc_body,
        grid=(GRID,),
        in_specs=[spec],
        out_specs=spec,
        out_shape=out,
    )(table)


def kernel(batch_size, query_embed_weight, query_pos_weight):
    query_pos = _sc_copy(query_pos_weight)
    query_embed = _tc_copy(query_embed_weight)
    return (query_embed, query_pos)
